# baseline (speedup 1.0000x reference)
import functools

import jax
import jax.numpy as jnp
from jax import lax
from jax.experimental import pallas as pl
from jax.experimental.pallas import tpu as pltpu
from jax.experimental.pallas import tpu_sc as plsc

NUM_RINGS = 50
EMBED_DIM = 64
FLAT = NUM_RINGS * EMBED_DIM
BATCH = 16384

NC = 2
NS = 16
LANES = 16
NW = NC * NS
ROWS_PER_W = BATCH // NW
CH = 16
NSTEPS = ROWS_PER_W // CH
NVREG = FLAT // LANES


def _sc_body(x_hbm, w_hbm, o_hbm, wv, b0, b1, si0, si1, so0, so1):
    cid = lax.axis_index("c")
    sid = lax.axis_index("s")
    wid = sid * NC + cid
    base = wid * ROWS_PER_W

    pltpu.sync_copy(w_hbm, wv)

    bufs = (b0, b1)
    isems = (si0, si1)
    osems = (so0, so1)
    in_h = [None, None]
    out_h = [None, None]

    in_h[0] = pltpu.async_copy(x_hbm.at[pl.ds(base, CH)], bufs[0], isems[0])

    for step in range(NSTEPS):
        k = step % 2
        nk = (step + 1) % 2
        if step + 1 < NSTEPS:
            if step >= 1:
                out_h[nk].wait()
            in_h[nk] = pltpu.async_copy(
                x_hbm.at[pl.ds(base + (step + 1) * CH, CH)], bufs[nk], isems[nk])
        in_h[k].wait()

        buf = bufs[k]

        def jbody(j, _, buf=buf):
            col = j * (2 * LANES)
            w16a = wv[pl.ds(col, LANES)]
            w16b = wv[pl.ds(col + LANES, LANES)]
            for cc in range(CH):
                buf[cc, pl.ds(col, LANES)] = buf[cc, pl.ds(col, LANES)] + w16a
                buf[cc, pl.ds(col + LANES, LANES)] = (
                    buf[cc, pl.ds(col + LANES, LANES)] + w16b)
            return 0

        lax.fori_loop(0, NVREG // 2, jbody, 0)

        out_h[k] = pltpu.async_copy(
            buf, o_hbm.at[pl.ds(base + step * CH, CH)], osems[k])

    out_h[0].wait()
    out_h[1].wait()


def kernel(x, W):
    B = x.shape[0]
    xf = x.reshape(B, FLAT)
    wf = W.reshape(FLAT)
    mesh = plsc.VectorSubcoreMesh(core_axis_name="c", subcore_axis_name="s")
    out = pl.kernel(
        _sc_body,
        out_type=jax.ShapeDtypeStruct((BATCH, FLAT), jnp.float32),
        mesh=mesh,
        scratch_types=[
            pltpu.VMEM((FLAT,), jnp.float32),
            pltpu.VMEM((CH, FLAT), jnp.float32),
            pltpu.VMEM((CH, FLAT), jnp.float32),
            pltpu.SemaphoreType.DMA,
            pltpu.SemaphoreType.DMA,
            pltpu.SemaphoreType.DMA,
            pltpu.SemaphoreType.DMA,
        ],
    )(xf, wf)
    return out.reshape(B, NUM_RINGS, EMBED_DIM)


# speedup vs baseline: 1.3359x; 1.3359x over previous
import functools

import jax
import jax.numpy as jnp
from jax import lax
from jax.experimental import pallas as pl
from jax.experimental.pallas import tpu as pltpu
from jax.experimental.pallas import tpu_sc as plsc

NUM_RINGS = 50
EMBED_DIM = 64
FLAT = NUM_RINGS * EMBED_DIM
BATCH = 16384

NC = 2
NS = 16
LANES = 16
NW = NC * NS
ROWS_PER_W = BATCH // NW
CH = 16
NSTEPS = ROWS_PER_W // CH
NVREG = FLAT // LANES


def _sc_body(x_hbm, w_hbm, o_hbm, wv, b0, b1, si0, si1, so0, so1):
    cid = lax.axis_index("c")
    sid = lax.axis_index("s")
    wid = sid * NC + cid
    base = wid * ROWS_PER_W

    pltpu.sync_copy(w_hbm, wv)

    bufs = (b0, b1)
    isems = (si0, si1)
    osems = (so0, so1)
    in_h = [None, None]
    out_h = [None, None]

    in_h[0] = pltpu.async_copy(x_hbm.at[pl.ds(base, CH)], bufs[0], isems[0])

    for step in range(NSTEPS):
        k = step % 2
        nk = (step + 1) % 2
        if step + 1 < NSTEPS:
            if step >= 1:
                out_h[nk].wait()
            in_h[nk] = pltpu.async_copy(
                x_hbm.at[pl.ds(base + (step + 1) * CH, CH)], bufs[nk], isems[nk])
        in_h[k].wait()

        buf = bufs[k]

        def jbody(j, _, buf=buf):
            w16 = wv[pl.ds(j * LANES, LANES)]
            for cc in range(CH):
                buf[cc, pl.ds(j * LANES, LANES)] = (
                    buf[cc, pl.ds(j * LANES, LANES)] + w16)
            return 0

        lax.fori_loop(0, NVREG, jbody, 0)

        out_h[k] = pltpu.async_copy(
            buf, o_hbm.at[pl.ds(base + step * CH, CH)], osems[k])

    out_h[0].wait()
    out_h[1].wait()


def kernel(x, W):
    B = x.shape[0]
    xf = x.reshape(B, FLAT)
    wf = W.reshape(FLAT)
    mesh = plsc.VectorSubcoreMesh(core_axis_name="c", subcore_axis_name="s")
    out = pl.kernel(
        _sc_body,
        out_type=jax.ShapeDtypeStruct((BATCH, FLAT), jnp.float32),
        mesh=mesh,
        scratch_types=[
            pltpu.VMEM((FLAT,), jnp.float32),
            pltpu.VMEM((CH, FLAT), jnp.float32),
            pltpu.VMEM((CH, FLAT), jnp.float32),
            pltpu.SemaphoreType.DMA,
            pltpu.SemaphoreType.DMA,
            pltpu.SemaphoreType.DMA,
            pltpu.SemaphoreType.DMA,
        ],
    )(xf, wf)
    return out.reshape(B, NUM_RINGS, EMBED_DIM)


# speedup vs baseline: 1.4685x; 1.0993x over previous
import functools

import jax
import jax.numpy as jnp
from jax import lax
from jax.experimental import pallas as pl
from jax.experimental.pallas import tpu as pltpu
from jax.experimental.pallas import tpu_sc as plsc

NUM_RINGS = 50
EMBED_DIM = 64
FLAT = NUM_RINGS * EMBED_DIM
BATCH = 16384

NC = 2
NS = 16
LANES = 16
NW = NC * NS
ROWS_PER_W = BATCH // NW
CH = 16
NSTEPS = ROWS_PER_W // CH
NVREG = FLAT // LANES


def _sc_body(x_hbm, w_hbm, o_hbm, wv, b0, b1, si0, si1, so0, so1):
    cid = lax.axis_index("c")
    sid = lax.axis_index("s")
    wid = sid * NC + cid
    base = wid * ROWS_PER_W

    pltpu.sync_copy(w_hbm, wv)

    bufs = (b0, b1)
    isems = (si0, si1)
    osems = (so0, so1)
    in_h = [None, None]
    out_h = [None, None]

    in_h[0] = pltpu.async_copy(x_hbm.at[pl.ds(base, CH)], bufs[0], isems[0])

    for step in range(NSTEPS):
        k = step % 2
        nk = (step + 1) % 2
        if step + 1 < NSTEPS:
            if step >= 1:
                out_h[nk].wait()
            in_h[nk] = pltpu.async_copy(
                x_hbm.at[pl.ds(base + (step + 1) * CH, CH)], bufs[nk], isems[nk])
        in_h[k].wait()

        buf = bufs[k]

        @functools.partial(plsc.parallel_loop, 0, NVREG)
        def _(j, buf=buf):
            w16 = wv[pl.ds(j * LANES, LANES)]
            for cc in range(CH):
                buf[cc, pl.ds(j * LANES, LANES)] = (
                    buf[cc, pl.ds(j * LANES, LANES)] + w16)

        out_h[k] = pltpu.async_copy(
            buf, o_hbm.at[pl.ds(base + step * CH, CH)], osems[k])

    out_h[0].wait()
    out_h[1].wait()


def kernel(x, W):
    B = x.shape[0]
    xf = x.reshape(B, FLAT)
    wf = W.reshape(FLAT)
    mesh = plsc.VectorSubcoreMesh(core_axis_name="c", subcore_axis_name="s")
    out = pl.kernel(
        _sc_body,
        out_type=jax.ShapeDtypeStruct((BATCH, FLAT), jnp.float32),
        mesh=mesh,
        scratch_types=[
            pltpu.VMEM((FLAT,), jnp.float32),
            pltpu.VMEM((CH, FLAT), jnp.float32),
            pltpu.VMEM((CH, FLAT), jnp.float32),
            pltpu.SemaphoreType.DMA,
            pltpu.SemaphoreType.DMA,
            pltpu.SemaphoreType.DMA,
            pltpu.SemaphoreType.DMA,
        ],
    )(xf, wf)
    return out.reshape(B, NUM_RINGS, EMBED_DIM)
